# baseline (speedup 1.0000x reference)
import functools

import jax
import jax.numpy as jnp
from jax import lax
from jax.experimental import pallas as pl
from jax.experimental.pallas import tpu as pltpu
from jax.experimental.pallas import tpu_sc as plsc

N_NODES = 10000
DIM = 256
HALF = 128
N_EDGES = 160000
NC = 2
NS = 16

CHUNK = 40
EDGES_PER_TILE = N_EDGES // NS
CHUNKS_PER_TILE = EDGES_PER_TILE // CHUNK
NBUF = 6
FULL_ROUNDS = CHUNKS_PER_TILE // NBUF
TAIL = CHUNKS_PER_TILE - FULL_ROUNDS * NBUF

WB_ROWS = 640
WB_LAST = N_NODES - 15 * WB_ROWS

_sc_mesh = plsc.VectorSubcoreMesh(
    core_axis_name="c", subcore_axis_name="s", num_cores=NC, num_subcores=NS
)


@functools.partial(
    pl.kernel,
    out_type=(
        jax.ShapeDtypeStruct((N_NODES, HALF), jnp.float32),
        jax.ShapeDtypeStruct((N_NODES, HALF), jnp.float32),
    ),
    mesh=_sc_mesh,
    scratch_types=(
        [
            pltpu.VMEM_SHARED((N_NODES, HALF), jnp.float32),
            pltpu.VMEM((EDGES_PER_TILE,), jnp.int32),
        ]
        + [pltpu.VMEM((CHUNK, HALF), jnp.float32)] * NBUF
        + [pltpu.VMEM((CHUNK,), jnp.int32)] * NBUF
        + [pltpu.SemaphoreType.DMA] * (3 * NBUF)
        + [pltpu.VMEM((CHUNK, HALF), jnp.float32),
           pltpu.SemaphoreType.DMA]
    ),
)
def _sc_aggregate(x_lo, x_hi, row_hbm, col_hbm,
                  agg_lo, agg_hi, acc, col1d, *bufs_and_sems):
    rows = bufs_and_sems[:NBUF]
    rbufs = bufs_and_sems[NBUF:2 * NBUF]
    rsems = bufs_and_sems[2 * NBUF:3 * NBUF]
    gsems = bufs_and_sems[3 * NBUF:4 * NBUF]
    ssems = bufs_and_sems[4 * NBUF:5 * NBUF]
    zbuf, zsem = bufs_and_sems[5 * NBUF:]
    c = lax.axis_index("c")
    s = lax.axis_index("s")
    ebase = s * EDGES_PER_TILE

    pltpu.sync_copy(col_hbm.at[pl.ds(ebase, EDGES_PER_TILE)], col1d)

    def each_tile_static(fn):
        for t in range(NS):
            lo = t * WB_ROWS if t < NS - 1 else 15 * WB_ROWS
            n = WB_ROWS if t < NS - 1 else WB_LAST

            @pl.when(s == t)
            def _(lo=lo, n=n):
                fn(lo, n)

    def zero_zbuf(r, _):
        for t in range(HALF // 16):
            zbuf[r, pl.ds(t * 16, 16)] = jnp.zeros((16,), jnp.float32)
        return 0

    lax.fori_loop(0, CHUNK, zero_zbuf, 0)

    def zero_fill(lo, n):
        for j in range(n // CHUNK):
            pltpu.async_copy(zbuf, acc.at[pl.ds(lo + j * CHUNK, CHUNK)], zsem)

    each_tile_static(zero_fill)

    def run(x_half):
        def issue(k, b):
            pltpu.async_copy(row_hbm.at[pl.ds(ebase + k * CHUNK, CHUNK)],
                             rbufs[b], rsems[b])
            pltpu.async_copy(x_half.at[col1d.at[pl.ds(k * CHUNK, CHUNK)]],
                             rows[b], gsems[b])

        def swait(b):
            pltpu.make_async_copy(rows[b], acc.at[rbufs[b]], ssems[b]).wait()

        def consume(k, b):
            pltpu.make_async_copy(x_half.at[col1d.at[pl.ds(0, CHUNK)]],
                                  rows[b], gsems[b]).wait()
            pltpu.make_async_copy(row_hbm.at[pl.ds(0, CHUNK)],
                                  rbufs[b], rsems[b]).wait()
            pltpu.async_copy(rows[b], acc.at[rbufs[b]], ssems[b], add=True)
            pb = (b - 1) % NBUF

            @pl.when(k >= 1)
            def _():
                swait(pb)

                @pl.when(k - 1 + NBUF < CHUNKS_PER_TILE)
                def _():
                    issue(k - 1 + NBUF, pb)

        for b in range(NBUF):
            issue(b, b)

        def zero_drain(lo, n):
            for _ in range(n // CHUNK):
                pltpu.make_async_copy(zbuf, acc.at[pl.ds(lo, CHUNK)],
                                      zsem).wait()

        each_tile_static(zero_drain)
        plsc.subcore_barrier()

        def outer(g, _):
            for b in range(NBUF):
                consume(g * NBUF + b, b)
            return 0

        lax.fori_loop(0, FULL_ROUNDS, outer, 0)
        for b in range(TAIL):
            consume(FULL_ROUNDS * NBUF + b, b)
        swait((CHUNKS_PER_TILE - 1) % NBUF)

    @pl.when(c == 0)
    def _():
        run(x_lo)

    @pl.when(c == 1)
    def _():
        run(x_hi)

    plsc.subcore_barrier()

    @pl.when(c == 0)
    def _():
        each_tile_static(lambda lo, n: pltpu.sync_copy(
            acc.at[pl.ds(lo, n)], agg_lo.at[pl.ds(lo, n)]))

    @pl.when(c == 1)
    def _():
        each_tile_static(lambda lo, n: pltpu.sync_copy(
            acc.at[pl.ds(lo, n)], agg_hi.at[pl.ds(lo, n)]))


BR = 2000


def _tc_pre_body(x_ref, ws_ref, bs_ref, bn_ref, h_ref):
    dn = (((1,), (1,)), ((), ()))
    h = lax.dot_general(x_ref[...], ws_ref[...], dn,
                        preferred_element_type=jnp.float32)
    h_ref[...] = (h + bs_ref[...] + bn_ref[...]).astype(jnp.bfloat16)


_tc_pre = pl.pallas_call(
    _tc_pre_body,
    grid=(N_NODES // BR,),
    in_specs=[
        pl.BlockSpec((BR, DIM), lambda i: (i, 0)),
        pl.BlockSpec((DIM, DIM), lambda i: (0, 0)),
        pl.BlockSpec((1, DIM), lambda i: (0, 0)),
        pl.BlockSpec((1, DIM), lambda i: (0, 0)),
    ],
    out_specs=pl.BlockSpec((BR, DIM), lambda i: (i, 0)),
    out_shape=jax.ShapeDtypeStruct((N_NODES, DIM), jnp.bfloat16),
)


def _tc_post_body(h_ref, alo_ref, ahi_ref, wn_ref, o_ref):
    dn = (((1,), (1,)), ((), ()))
    wn = wn_ref[...]
    h = h_ref[...].astype(jnp.float32)
    h = h + lax.dot_general(alo_ref[...], wn[:, :HALF], dn,
                            preferred_element_type=jnp.float32)
    h = h + lax.dot_general(ahi_ref[...], wn[:, HALF:], dn,
                            preferred_element_type=jnp.float32)
    o_ref[...] = jnp.maximum(h, 0.0)


_tc_post = pl.pallas_call(
    _tc_post_body,
    grid=(N_NODES // BR,),
    in_specs=[
        pl.BlockSpec((BR, DIM), lambda i: (i, 0)),
        pl.BlockSpec((BR, HALF), lambda i: (i, 0)),
        pl.BlockSpec((BR, HALF), lambda i: (i, 0)),
        pl.BlockSpec((DIM, DIM), lambda i: (0, 0)),
    ],
    out_specs=pl.BlockSpec((BR, DIM), lambda i: (i, 0)),
    out_shape=jax.ShapeDtypeStruct((N_NODES, DIM), jnp.float32),
)


def kernel(x, edge_index, W_self, b_self, W_neigh, b_neigh):
    ei = edge_index.astype(jnp.int32)
    x_lo = x[:, :HALF]
    x_hi = x[:, HALF:]
    agg_lo, agg_hi = _sc_aggregate(x_lo, x_hi, ei[0], ei[1])
    h = _tc_pre(x, W_self, b_self.reshape(1, DIM), b_neigh.reshape(1, DIM))
    return _tc_post(h, agg_lo, agg_hi, W_neigh)


# speedup vs baseline: 1.0130x; 1.0130x over previous
import functools

import jax
import jax.numpy as jnp
from jax import lax
from jax.experimental import pallas as pl
from jax.experimental.pallas import tpu as pltpu
from jax.experimental.pallas import tpu_sc as plsc

N_NODES = 10000
DIM = 256
HALF = 128
N_EDGES = 160000
NC = 2
NS = 16

CHUNK = 40
EDGES_PER_TILE = N_EDGES // NS
CHUNKS_PER_TILE = EDGES_PER_TILE // CHUNK
NBUF = 6
FULL_ROUNDS = CHUNKS_PER_TILE // NBUF
TAIL = CHUNKS_PER_TILE - FULL_ROUNDS * NBUF

WB_ROWS = 640
WB_LAST = N_NODES - 15 * WB_ROWS

_sc_mesh = plsc.VectorSubcoreMesh(
    core_axis_name="c", subcore_axis_name="s", num_cores=NC, num_subcores=NS
)


@functools.partial(
    pl.kernel,
    out_type=(
        jax.ShapeDtypeStruct((N_NODES, HALF), jnp.float32),
        jax.ShapeDtypeStruct((N_NODES, HALF), jnp.float32),
    ),
    mesh=_sc_mesh,
    scratch_types=(
        [
            pltpu.VMEM_SHARED((N_NODES, HALF), jnp.float32),
            pltpu.VMEM((EDGES_PER_TILE,), jnp.int32),
        ]
        + [pltpu.VMEM((CHUNK, HALF), jnp.float32)] * NBUF
        + [pltpu.VMEM((CHUNK,), jnp.int32)] * NBUF
        + [pltpu.SemaphoreType.DMA] * (3 * NBUF)
        + [pltpu.VMEM((CHUNK, HALF), jnp.float32),
           pltpu.SemaphoreType.DMA]
    ),
)
def _sc_aggregate(x_lo, x_hi, row_hbm, col_hbm,
                  agg_lo, agg_hi, acc, col1d, *bufs_and_sems):
    rows = bufs_and_sems[:NBUF]
    rbufs = bufs_and_sems[NBUF:2 * NBUF]
    rsems = bufs_and_sems[2 * NBUF:3 * NBUF]
    gsems = bufs_and_sems[3 * NBUF:4 * NBUF]
    ssems = bufs_and_sems[4 * NBUF:5 * NBUF]
    zbuf, zsem = bufs_and_sems[5 * NBUF:]
    c = lax.axis_index("c")
    s = lax.axis_index("s")
    ebase = s * EDGES_PER_TILE

    pltpu.sync_copy(col_hbm.at[pl.ds(ebase, EDGES_PER_TILE)], col1d)

    def each_tile_static(fn):
        for t in range(NS):
            lo = t * WB_ROWS if t < NS - 1 else 15 * WB_ROWS
            n = WB_ROWS if t < NS - 1 else WB_LAST

            @pl.when(s == t)
            def _(lo=lo, n=n):
                fn(lo, n)

    def zero_zbuf(r, _):
        for t in range(HALF // 16):
            zbuf[r, pl.ds(t * 16, 16)] = jnp.zeros((16,), jnp.float32)
        return 0

    lax.fori_loop(0, CHUNK, zero_zbuf, 0)

    def zero_fill(lo, n):
        for j in range(n // CHUNK):
            pltpu.async_copy(zbuf, acc.at[pl.ds(lo + j * CHUNK, CHUNK)], zsem)

    each_tile_static(zero_fill)

    def run(x_half):
        def issue(k, b):
            pltpu.async_copy(row_hbm.at[pl.ds(ebase + k * CHUNK, CHUNK)],
                             rbufs[b], rsems[b])
            pltpu.async_copy(x_half.at[col1d.at[pl.ds(k * CHUNK, CHUNK)]],
                             rows[b], gsems[b])

        def consume(k, b):
            pltpu.make_async_copy(x_half.at[col1d.at[pl.ds(0, CHUNK)]],
                                  rows[b], gsems[b]).wait()
            pltpu.make_async_copy(row_hbm.at[pl.ds(0, CHUNK)],
                                  rbufs[b], rsems[b]).wait()
            pltpu.async_copy(rows[b], acc.at[rbufs[b]], ssems[b], add=True)
            pltpu.make_async_copy(rows[b], acc.at[rbufs[b]], ssems[b]).wait()

            @pl.when(k + NBUF < CHUNKS_PER_TILE)
            def _():
                issue(k + NBUF, b)

        for b in range(NBUF):
            issue(b, b)

        def zero_drain(lo, n):
            for _ in range(n // CHUNK):
                pltpu.make_async_copy(zbuf, acc.at[pl.ds(lo, CHUNK)],
                                      zsem).wait()

        each_tile_static(zero_drain)
        plsc.subcore_barrier()

        def outer(g, _):
            for b in range(NBUF):
                consume(g * NBUF + b, b)
            return 0

        lax.fori_loop(0, FULL_ROUNDS, outer, 0)
        for b in range(TAIL):
            consume(FULL_ROUNDS * NBUF + b, b)

    @pl.when(c == 0)
    def _():
        run(x_lo)

    @pl.when(c == 1)
    def _():
        run(x_hi)

    plsc.subcore_barrier()

    @pl.when(c == 0)
    def _():
        each_tile_static(lambda lo, n: pltpu.sync_copy(
            acc.at[pl.ds(lo, n)], agg_lo.at[pl.ds(lo, n)]))

    @pl.when(c == 1)
    def _():
        each_tile_static(lambda lo, n: pltpu.sync_copy(
            acc.at[pl.ds(lo, n)], agg_hi.at[pl.ds(lo, n)]))


BR = 2000


def _tc_pre_body(x_ref, ws_ref, bs_ref, bn_ref, h_ref):
    dn = (((1,), (1,)), ((), ()))
    h = lax.dot_general(x_ref[...], ws_ref[...], dn,
                        preferred_element_type=jnp.float32)
    h_ref[...] = (h + bs_ref[...] + bn_ref[...]).astype(jnp.bfloat16)


_tc_pre = pl.pallas_call(
    _tc_pre_body,
    grid=(N_NODES // BR,),
    in_specs=[
        pl.BlockSpec((BR, DIM), lambda i: (i, 0)),
        pl.BlockSpec((DIM, DIM), lambda i: (0, 0)),
        pl.BlockSpec((1, DIM), lambda i: (0, 0)),
        pl.BlockSpec((1, DIM), lambda i: (0, 0)),
    ],
    out_specs=pl.BlockSpec((BR, DIM), lambda i: (i, 0)),
    out_shape=jax.ShapeDtypeStruct((N_NODES, DIM), jnp.bfloat16),
)


def _tc_post_body(h_ref, alo_ref, ahi_ref, wn_ref, o_ref):
    dn = (((1,), (1,)), ((), ()))
    wn = wn_ref[...]
    h = h_ref[...].astype(jnp.float32)
    h = h + lax.dot_general(alo_ref[...], wn[:, :HALF], dn,
                            preferred_element_type=jnp.float32)
    h = h + lax.dot_general(ahi_ref[...], wn[:, HALF:], dn,
                            preferred_element_type=jnp.float32)
    o_ref[...] = jnp.maximum(h, 0.0)


_tc_post = pl.pallas_call(
    _tc_post_body,
    grid=(N_NODES // BR,),
    in_specs=[
        pl.BlockSpec((BR, DIM), lambda i: (i, 0)),
        pl.BlockSpec((BR, HALF), lambda i: (i, 0)),
        pl.BlockSpec((BR, HALF), lambda i: (i, 0)),
        pl.BlockSpec((DIM, DIM), lambda i: (0, 0)),
    ],
    out_specs=pl.BlockSpec((BR, DIM), lambda i: (i, 0)),
    out_shape=jax.ShapeDtypeStruct((N_NODES, DIM), jnp.float32),
)


def kernel(x, edge_index, W_self, b_self, W_neigh, b_neigh):
    ei = edge_index.astype(jnp.int32)
    x_lo = x[:, :HALF]
    x_hi = x[:, HALF:]
    agg_lo, agg_hi = _sc_aggregate(x_lo, x_hi, ei[0], ei[1])
    h = _tc_pre(x, W_self, b_self.reshape(1, DIM), b_neigh.reshape(1, DIM))
    return _tc_post(h, agg_lo, agg_hi, W_neigh)
